# baseline (speedup 1.0000x reference)
import functools

import jax
import jax.numpy as jnp
from jax import lax
from jax.experimental import pallas as pl
from jax.experimental.pallas import tpu as pltpu
from jax.experimental.pallas import tpu_sc as plsc

G = 256
NROWS, NCOLS = 4096, 512
NPTS = NROWS * NCOLS
NC, NS, L = 2, 16, 16
NW = NC * NS
PTS_PER_W = NPTS // NW
P = 4096
N_CHUNKS = PTS_PER_W // P
PLANES_PER_TILE = G // NS
WORDS_PER_PLANE = G * G // 32


@functools.partial(
    pl.kernel,
    out_type=jax.ShapeDtypeStruct((NPTS,), jnp.int32),
    mesh=plsc.VectorSubcoreMesh(core_axis_name="c", subcore_axis_name="s"),
    compiler_params=pltpu.CompilerParams(
        needs_layout_passes=False, use_tc_tiling_on_sc=False
    ),
    scratch_types=[
        pltpu.VMEM_SHARED((G * G * G // 32,), jnp.int32),
        pltpu.VMEM((2, G, G), jnp.uint8),
        pltpu.VMEM((2, WORDS_PER_PLANE,), jnp.int32),
        pltpu.VMEM((2, 3, P), jnp.float32),
        pltpu.VMEM((2, P), jnp.int32),
        pltpu.VMEM((2, P), jnp.int32),
        pltpu.VMEM((2, P), jnp.int32),
        pltpu.VMEM((2, P), jnp.int32),
        pltpu.SemaphoreType.DMA,
        pltpu.SemaphoreType.DMA,
        pltpu.SemaphoreType.DMA,
        pltpu.SemaphoreType.DMA,
        pltpu.SemaphoreType.DMA,
        pltpu.SemaphoreType.DMA,
    ],
)
def _sc_prune(xyz_hbm, grid_hbm, out_hbm, tbl_sp, planev, wordbuf, posv,
              widxv, bidxv, wordsv, outv, psem, possem, gsem0, gsem1, osem,
              tsem):
    gsems = (gsem0, gsem1)
    cid = lax.axis_index("c")
    sid = lax.axis_index("s")
    wid = sid * NC + cid

    def pack_plane(pi, buf):
        p = sid * PLANES_PER_TILE + pi

        @plsc.parallel_loop(0, WORDS_PER_PLANE // L, unroll=8)
        def pack_vec(ov):
            r = ov * 2
            w = None
            for j in range(8):
                v8 = planev[buf, r + (j >> 2), pl.ds((j & 3) * 64, 64)]
                v = plsc.bitcast(v8, jnp.int32)
                nib = lax.shift_right_logical(v * 0x08040201, 24)
                w = nib if j == 0 else w | (nib << (4 * j))
            wordbuf[buf, pl.ds(ov * L, L)] = w

        return pltpu.async_copy(
            wordbuf.at[buf],
            tbl_sp.at[pl.ds(p * WORDS_PER_PLANE, WORDS_PER_PLANE)], tsem)

    def plane_dma(pi, buf):
        return pltpu.async_copy(
            grid_hbm.at[sid * PLANES_PER_TILE + pi], planev.at[buf], psem)

    first_pos = [
        pltpu.async_copy(xyz_hbm.at[c, pl.ds(wid * PTS_PER_W, P)],
                         posv.at[0, c], possem)
        for c in range(3)
    ]
    dma = plane_dma(0, 0)
    tput = None
    for pi in range(PLANES_PER_TILE):
        dma.wait()
        if pi + 1 < PLANES_PER_TILE:
            dma = plane_dma(pi + 1, (pi + 1) & 1)
        if tput is not None:
            tput.wait()
        tput = pack_plane(pi, pi & 1)
    tput.wait()
    plsc.subcore_barrier()

    def pos_dma(k, buf):
        base = wid * PTS_PER_W + k * P
        return [
            pltpu.async_copy(xyz_hbm.at[c, pl.ds(base, P)], posv.at[buf, c],
                             possem)
            for c in range(3)
        ]


    def compute_idx(k, buf):
        @plsc.parallel_loop(0, P // L, unroll=4)
        def _(g):
            s = pl.ds(g * L, L)
            ix = (posv[buf, 0, s] * float(G)).astype(jnp.int32)
            iy = (posv[buf, 1, s] * float(G)).astype(jnp.int32)
            iz = (posv[buf, 2, s] * float(G)).astype(jnp.int32)
            lin = (ix << 16) | (iy << 8) | iz
            widxv[buf, s] = (lax.shift_right_logical(lin, 5) & -16) | (
                lax.shift_right_logical(lin, 2) & 15)
            bidxv[buf, s] = (lax.shift_right_logical(lin, 4) & 28) | (
                (lin & 3) ^ 3)

    def extract_bits(k, buf):
        @plsc.parallel_loop(0, P // L, unroll=4)
        def _(g):
            s = pl.ds(g * L, L)
            outv[buf, s] = (
                lax.shift_right_logical(wordsv[buf, s], bidxv[buf, s]) & 1)
        return pltpu.async_copy(
            outv.at[buf], out_hbm.at[pl.ds(wid * PTS_PER_W + k * P, P)], osem)

    pdmas = first_pos
    gdma = None
    odmas = [None, None]
    for k in range(N_CHUNKS):
        b = k & 1
        for d in pdmas:
            d.wait()
        if k + 1 < N_CHUNKS:
            pdmas = pos_dma(k + 1, 1 - b)
        compute_idx(k, b)
        prev = gdma
        gdma = pltpu.async_copy(tbl_sp.at[widxv.at[b]], wordsv.at[b], gsems[b])
        if prev is not None:
            prev.wait()
            if odmas[b] is not None:
                odmas[b].wait()
            odmas[1 - b] = extract_bits(k - 1, 1 - b)
    gdma.wait()
    b = (N_CHUNKS - 1) & 1
    last = extract_bits(N_CHUNKS - 1, b)
    odmas[1 - b].wait()
    last.wait()


def kernel(positions, is_training, voxel_grid):
    xyz = jnp.moveaxis(positions, 2, 0).reshape(3, NPTS)
    out = _sc_prune(xyz, voxel_grid.astype(jnp.uint8))
    return out.reshape(NROWS, NCOLS).astype(jnp.bool_)


# speedup vs baseline: 1.0118x; 1.0118x over previous
import functools

import jax
import jax.numpy as jnp
from jax import lax
from jax.experimental import pallas as pl
from jax.experimental.pallas import tpu as pltpu
from jax.experimental.pallas import tpu_sc as plsc

G = 256
NROWS, NCOLS = 4096, 512
NPTS = NROWS * NCOLS
NC, NS, L = 2, 16, 16
NW = NC * NS
PTS_PER_W = NPTS // NW
P = 4096
N_CHUNKS = PTS_PER_W // P
PLANES_PER_TILE = G // NS
WORDS_PER_PLANE = G * G // 32


@functools.partial(
    pl.kernel,
    out_type=jax.ShapeDtypeStruct((NPTS,), jnp.int32),
    mesh=plsc.VectorSubcoreMesh(core_axis_name="c", subcore_axis_name="s"),
    compiler_params=pltpu.CompilerParams(
        needs_layout_passes=False, use_tc_tiling_on_sc=False
    ),
    scratch_types=[
        pltpu.VMEM_SHARED((G * G * G // 32,), jnp.int32),
        pltpu.VMEM((2, G, G), jnp.uint8),
        pltpu.VMEM((2, WORDS_PER_PLANE,), jnp.int32),
        pltpu.VMEM((2, 3, P), jnp.float32),
        pltpu.VMEM((2, P), jnp.int32),
        pltpu.VMEM((2, P), jnp.int32),
        pltpu.VMEM((2, P), jnp.int32),
        pltpu.VMEM((2, P), jnp.int32),
        pltpu.SemaphoreType.DMA,
        pltpu.SemaphoreType.DMA,
        pltpu.SemaphoreType.DMA,
        pltpu.SemaphoreType.DMA,
        pltpu.SemaphoreType.DMA,
        pltpu.SemaphoreType.DMA,
    ],
)
def _sc_prune(xyz_hbm, grid_hbm, out_hbm, tbl_sp, planev, wordbuf, posv,
              widxv, bidxv, wordsv, outv, psem, possem, gsem0, gsem1, osem,
              tsem):
    gsems = (gsem0, gsem1)
    cid = lax.axis_index("c")
    sid = lax.axis_index("s")
    wid = sid * NC + cid

    def pack_plane(pi, buf):
        p = sid * PLANES_PER_TILE + pi

        @plsc.parallel_loop(0, WORDS_PER_PLANE // L, unroll=4)
        def pack_vec(ov):
            r = ov * 2
            w = None
            for j in range(8):
                v8 = planev[buf, r + (j >> 2), pl.ds((j & 3) * 64, 64)]
                v = plsc.bitcast(v8, jnp.int32)
                nib = lax.shift_right_logical(v * 0x08040201, 24)
                w = nib if j == 0 else w | (nib << (4 * j))
            wordbuf[buf, pl.ds(ov * L, L)] = w

        return pltpu.async_copy(
            wordbuf.at[buf],
            tbl_sp.at[pl.ds(p * WORDS_PER_PLANE, WORDS_PER_PLANE)], tsem)

    def plane_dma(pi, buf):
        return pltpu.async_copy(
            grid_hbm.at[sid * PLANES_PER_TILE + pi], planev.at[buf], psem)

    first_pos = [
        pltpu.async_copy(xyz_hbm.at[c, pl.ds(wid * PTS_PER_W, P)],
                         posv.at[0, c], possem)
        for c in range(3)
    ]
    dma = plane_dma(0, 0)
    tput = None
    for pi in range(PLANES_PER_TILE):
        dma.wait()
        if pi + 1 < PLANES_PER_TILE:
            dma = plane_dma(pi + 1, (pi + 1) & 1)
        if tput is not None:
            tput.wait()
        tput = pack_plane(pi, pi & 1)
    tput.wait()
    plsc.subcore_barrier()

    def pos_dma(k, buf):
        base = wid * PTS_PER_W + k * P
        return [
            pltpu.async_copy(xyz_hbm.at[c, pl.ds(base, P)], posv.at[buf, c],
                             possem)
            for c in range(3)
        ]


    def compute_idx(k, buf):
        @plsc.parallel_loop(0, P // L, unroll=4)
        def _(g):
            s = pl.ds(g * L, L)
            ix = (posv[buf, 0, s] * float(G)).astype(jnp.int32)
            iy = (posv[buf, 1, s] * float(G)).astype(jnp.int32)
            iz = (posv[buf, 2, s] * float(G)).astype(jnp.int32)
            lin = (ix << 16) | (iy << 8) | iz
            widxv[buf, s] = (lax.shift_right_logical(lin, 5) & -16) | (
                lax.shift_right_logical(lin, 2) & 15)
            bidxv[buf, s] = (lax.shift_right_logical(lin, 4) & 28) | (
                (lin & 3) ^ 3)

    def extract_bits(k, buf):
        @plsc.parallel_loop(0, P // L, unroll=4)
        def _(g):
            s = pl.ds(g * L, L)
            outv[buf, s] = (
                lax.shift_right_logical(wordsv[buf, s], bidxv[buf, s]) & 1)
        return pltpu.async_copy(
            outv.at[buf], out_hbm.at[pl.ds(wid * PTS_PER_W + k * P, P)], osem)

    pdmas = first_pos
    gdma = None
    odmas = [None, None]
    for k in range(N_CHUNKS):
        b = k & 1
        for d in pdmas:
            d.wait()
        if k + 1 < N_CHUNKS:
            pdmas = pos_dma(k + 1, 1 - b)
        compute_idx(k, b)
        prev = gdma
        gdma = pltpu.async_copy(tbl_sp.at[widxv.at[b]], wordsv.at[b], gsems[b])
        if prev is not None:
            prev.wait()
            if odmas[b] is not None:
                odmas[b].wait()
            odmas[1 - b] = extract_bits(k - 1, 1 - b)
    gdma.wait()
    b = (N_CHUNKS - 1) & 1
    last = extract_bits(N_CHUNKS - 1, b)
    odmas[1 - b].wait()
    last.wait()


def kernel(positions, is_training, voxel_grid):
    xyz = jnp.moveaxis(positions, 2, 0).reshape(3, NPTS)
    out = _sc_prune(xyz, voxel_grid.astype(jnp.uint8))
    return out.reshape(NROWS, NCOLS).astype(jnp.bool_)
